# baseline (speedup 1.0000x reference)
import functools

import jax
import jax.numpy as jnp
from jax import lax
from jax.experimental import pallas as pl
from jax.experimental.pallas import tpu as pltpu
from jax.experimental.pallas import tpu_sc as plsc

EMB = 64
_NC = 2
_NS = 16
_NW = _NC * _NS


@functools.lru_cache(maxsize=None)
def _make_gather(B: int, n_chunks: int, chunk: int):
    b_per_w = B // _NW
    assert b_per_w == n_chunks * chunk

    mesh = plsc.VectorSubcoreMesh(core_axis_name="c", subcore_axis_name="s")

    @functools.partial(
        pl.kernel,
        mesh=mesh,
        out_type=jax.ShapeDtypeStruct((B, EMB), jnp.float32),
        scratch_types=[
            pltpu.VMEM((chunk,), jnp.int32),
            pltpu.VMEM((chunk,), jnp.int32),
            pltpu.VMEM((chunk, EMB), jnp.float32),
            pltpu.VMEM((chunk, EMB), jnp.float32),
            pltpu.SemaphoreType.DMA,
            pltpu.SemaphoreType.DMA,
        ],
        compiler_params=pltpu.CompilerParams(use_tc_tiling_on_sc=False),
    )
    def k(idx_hbm, table_hbm, out_hbm, idx0, idx1, rows0, rows1, gsem, osem):
        wid = lax.axis_index("s") * _NC + lax.axis_index("c")
        base = wid * b_per_w
        idx_v = (idx0, idx1)
        rows_v = (rows0, rows1)

        def idx_src(j):
            return idx_hbm.at[pl.ds(base + j * chunk, chunk)]

        def out_dst(j):
            return out_hbm.at[pl.ds(base + j * chunk, chunk)]

        pltpu.sync_copy(idx_src(0), idx0)
        pltpu.async_copy(table_hbm.at[idx0], rows0, gsem)
        for j in range(n_chunks):
            cur, nxt = j % 2, (j + 1) % 2
            if j + 1 < n_chunks:
                pltpu.sync_copy(idx_src(j + 1), idx_v[nxt])
                if j >= 1:
                    pltpu.make_async_copy(
                        rows_v[nxt], out_dst(j - 1), osem
                    ).wait()
                pltpu.async_copy(table_hbm.at[idx_v[nxt]], rows_v[nxt], gsem)
            pltpu.make_async_copy(
                table_hbm.at[idx_v[cur]], rows_v[cur], gsem
            ).wait()
            pltpu.async_copy(rows_v[cur], out_dst(j), osem)
        for j in (n_chunks - 2, n_chunks - 1):
            pltpu.make_async_copy(rows_v[j % 2], out_dst(j), osem).wait()

    return k


def kernel(input_ids, table):
    B = input_ids.shape[0] * input_ids.shape[1]
    ids_flat = input_ids.reshape(-1).astype(jnp.int32)
    out = _make_gather(B, 10, B // _NW // 10)(ids_flat, table)
    return out.reshape(input_ids.shape + (EMB,))
